# baseline (speedup 1.0000x reference)
import functools
import math

import jax
import jax.numpy as jnp
from jax import lax
from jax.experimental import pallas as pl
from jax.experimental.pallas import tpu as pltpu
from jax.experimental.pallas import tpu_sc as plsc

S = 64.0
M1 = 1.0
M2 = 0.5
M3 = 0.0
ALPHA = 0.1

_R = 256
_K = 4096
_NEG = -3.0e38


def _acos(x):
    return jnp.arctan2(jnp.sqrt(jnp.maximum((1.0 - x) * (1.0 + x), 0.0)), x)


def _pass_a_body(C, CB, labels_ref, x_ref, out_ref, vals_ref, accmax, acccos):
    j = pl.program_id(1)
    x = x_ref[...]
    out_ref[...] = x * S
    lbl = labels_ref[0, 0, :]
    safe = jnp.where(lbl == -1, 0, lbl)
    cols = lax.broadcasted_iota(jnp.int32, (_R, _K), 1) + j * _K
    is_lbl = cols == safe[:, None]
    invalid = cols >= C
    mx = jnp.max(jnp.where(is_lbl | invalid, _NEG, x), axis=1)
    cs = jnp.max(jnp.where(is_lbl, x, _NEG), axis=1)

    @pl.when(j == 0)
    def _():
        accmax[...] = mx
        acccos[...] = cs

    @pl.when(j > 0)
    def _():
        accmax[...] = jnp.maximum(accmax[...], mx)
        acccos[...] = jnp.maximum(acccos[...], cs)

    @pl.when(j == CB - 1)
    def _():
        cos_y = acccos[...]
        max_o = accmax[...]
        ty = _acos(jnp.clip(cos_y, -1.0, 1.0))
        tm = _acos(jnp.clip(max_o, -1.0, 1.0))
        h = jnp.clip(jnp.float32(math.pi / 2) - (tm - ty),
                     0.0, jnp.float32(math.pi / 3))
        m_i = M2 + ALPHA * h
        phi = jnp.cos(M1 * ty + m_i) - M3
        fin = jnp.where(phi < cos_y, phi, cos_y)
        val = jnp.where(lbl == -1, cos_y, fin) * S
        vals_ref[0, 0, :] = val


def _make_pass_a(B, C, interpret=False):
    RB = B // _R
    CB = (C + _K - 1) // _K
    return pl.pallas_call(
        functools.partial(_pass_a_body, C, CB),
        grid=(RB, CB),
        in_specs=[
            pl.BlockSpec((1, 1, _R), lambda i, j: (i, 0, 0)),
            pl.BlockSpec((_R, _K), lambda i, j: (i, j)),
        ],
        out_specs=[
            pl.BlockSpec((_R, _K), lambda i, j: (i, j)),
            pl.BlockSpec((1, 1, _R), lambda i, j: (i, 0, 0)),
        ],
        out_shape=[
            jax.ShapeDtypeStruct((B, C), jnp.float32),
            jax.ShapeDtypeStruct((RB, 1, _R), jnp.float32),
        ],
        scratch_shapes=[
            pltpu.VMEM((_R,), jnp.float32),
            pltpu.VMEM((_R,), jnp.float32),
        ],
        compiler_params=pltpu.CompilerParams(
            dimension_semantics=("parallel", "arbitrary"),
        ),
        interpret=interpret,
    )


def _make_sc_scatter(B):
    info = plsc.get_sparse_core_info()
    NC, NS = info.num_cores, info.num_subcores
    nw = NC * NS
    per_w = B // nw
    mesh = plsc.VectorSubcoreMesh(core_axis_name="c", subcore_axis_name="s")

    @functools.partial(
        pl.kernel,
        mesh=mesh,
        out_type=(),
        scratch_types=[
            pltpu.VMEM((per_w,), jnp.int32),
            pltpu.VMEM((per_w,), jnp.float32),
            pltpu.SemaphoreType.DMA,
        ],
    )
    def scatter(out_hbm, vals_hbm, idx_hbm, idx_v, vals_v, sem):
        wid = lax.axis_index("s") * NC + lax.axis_index("c")
        base = wid * per_w
        pltpu.sync_copy(idx_hbm.at[pl.ds(base, per_w)], idx_v)
        pltpu.sync_copy(vals_hbm.at[pl.ds(base, per_w)], vals_v)
        pltpu.async_copy(vals_v, out_hbm.at[idx_v], sem).wait()

    return scatter


def kernel(logits, labels):
    B, C = logits.shape
    out, vals3 = _make_pass_a(B, C)(labels.reshape(B // _R, 1, _R), logits)
    vals = vals3.reshape(B)
    safe = jnp.where(labels == -1, 0, labels)
    idx = jnp.arange(B, dtype=jnp.int32) * C + safe
    out_ref = jax.new_ref(out.reshape(B * C))
    _make_sc_scatter(B)(out_ref, vals, idx)
    return out_ref[...].reshape(B, C)


# speedup vs baseline: 7.5606x; 7.5606x over previous
import functools
import math

import jax
import jax.numpy as jnp
from jax import lax
from jax.experimental import pallas as pl
from jax.experimental.pallas import tpu as pltpu
from jax.experimental.pallas import tpu_sc as plsc

S = 64.0
M1 = 1.0
M2 = 0.5
M3 = 0.0
ALPHA = 0.1

_KR = 256
_NEG = -3.0e38


def _acos(x):
    return jnp.arctan2(jnp.sqrt(jnp.maximum((1.0 - x) * (1.0 + x), 0.0)), x)


def _pass_a_body(C, CB, labels_ref, x_ref, out_ref, vals_ref, accmax, acccos):
    j = pl.program_id(0)
    x = x_ref[...]
    out_ref[...] = x * S
    lbl = labels_ref[0, :, 0, :]
    safe = jnp.where(lbl == -1, 0, lbl)
    cls = (lax.broadcasted_iota(jnp.int32, (_KR, 8, 8, 128), 0) + j * _KR) * 8 \
        + lax.broadcasted_iota(jnp.int32, (_KR, 8, 8, 128), 2)
    is_lbl = cls == safe[None, :, None, :]
    invalid = cls >= C
    mx = jnp.max(jnp.where(is_lbl | invalid, _NEG, x), axis=(0, 2))
    cs = jnp.max(jnp.where(is_lbl, x, _NEG), axis=(0, 2))

    @pl.when(j == 0)
    def _():
        accmax[...] = mx
        acccos[...] = cs

    @pl.when(j > 0)
    def _():
        accmax[...] = jnp.maximum(accmax[...], mx)
        acccos[...] = jnp.maximum(acccos[...], cs)

    @pl.when(j == CB - 1)
    def _():
        cos_y = acccos[...]
        max_o = accmax[...]
        ty = _acos(jnp.clip(cos_y, -1.0, 1.0))
        tm = _acos(jnp.clip(max_o, -1.0, 1.0))
        h = jnp.clip(jnp.float32(math.pi / 2) - (tm - ty),
                     0.0, jnp.float32(math.pi / 3))
        m_i = M2 + ALPHA * h
        phi = jnp.cos(M1 * ty + m_i) - M3
        fin = jnp.where(phi < cos_y, phi, cos_y)
        vals_ref[...] = jnp.where(lbl == -1, cos_y, fin) * S


def _make_pass_a(RT, C, interpret=False):
    CB = (RT + _KR - 1) // _KR
    return pl.pallas_call(
        functools.partial(_pass_a_body, C, CB),
        grid=(CB,),
        in_specs=[
            pl.BlockSpec((1, 8, 1, 128), lambda j: (0, 0, 0, 0)),
            pl.BlockSpec((_KR, 8, 8, 128), lambda j: (j, 0, 0, 0)),
        ],
        out_specs=[
            pl.BlockSpec((_KR, 8, 8, 128), lambda j: (j, 0, 0, 0)),
            pl.BlockSpec((8, 128), lambda j: (0, 0)),
        ],
        out_shape=[
            jax.ShapeDtypeStruct((RT, 8, 8, 128), jnp.float32),
            jax.ShapeDtypeStruct((8, 128), jnp.float32),
        ],
        scratch_shapes=[
            pltpu.VMEM((8, 128), jnp.float32),
            pltpu.VMEM((8, 128), jnp.float32),
        ],
        compiler_params=pltpu.CompilerParams(
            dimension_semantics=("arbitrary",),
            vmem_limit_bytes=100 * 1024 * 1024,
        ),
        interpret=interpret,
    )


def _make_sc_scatter(B):
    info = plsc.get_sparse_core_info()
    NC, NS = info.num_cores, info.num_subcores
    nw = NC * NS
    per_w = B // nw
    mesh = plsc.VectorSubcoreMesh(core_axis_name="c", subcore_axis_name="s")

    @functools.partial(
        pl.kernel,
        mesh=mesh,
        out_type=(),
        scratch_types=[
            pltpu.VMEM((per_w,), jnp.int32),
            pltpu.VMEM((per_w,), jnp.float32),
            pltpu.SemaphoreType.DMA,
        ],
    )
    def scatter(out_hbm, vals_hbm, idx_hbm, idx_v, vals_v, sem):
        wid = lax.axis_index("s") * NC + lax.axis_index("c")
        base = wid * per_w
        pltpu.sync_copy(idx_hbm.at[pl.ds(base, per_w)], idx_v)
        pltpu.sync_copy(vals_hbm.at[pl.ds(base, per_w)], vals_v)
        pltpu.async_copy(vals_v, out_hbm.at[idx_v], sem).wait()

    return scatter


def kernel(logits, labels):
    B, C = logits.shape
    RT = C // 8
    x4 = logits.T.reshape(RT, 8, B // 128, 128).transpose(0, 2, 1, 3)
    lbl4 = labels.reshape(1, B // 128, 1, 128)
    out4, vals2 = _make_pass_a(RT, C)(lbl4, x4)
    vals = vals2.reshape(B)

    safe = jnp.where(labels == -1, 0, labels)
    b = jnp.arange(B, dtype=jnp.int32)
    idx = (safe // 8) * (B * 8) + (b // 128) * 1024 + (safe % 8) * 128 + (b % 128)
    out_ref = jax.new_ref(out4.reshape(RT * 8 * B))
    _make_sc_scatter(B)(out_ref, vals, idx)
    return out_ref[...].reshape(RT, B // 128, 8, 128).transpose(0, 2, 1, 3).reshape(C, B).T


# speedup vs baseline: 7.6014x; 1.0054x over previous
import functools
import math

import jax
import jax.numpy as jnp
from jax import lax
from jax.experimental import pallas as pl
from jax.experimental.pallas import tpu as pltpu
from jax.experimental.pallas import tpu_sc as plsc

S = 64.0
M1 = 1.0
M2 = 0.5
M3 = 0.0
ALPHA = 0.1

_KR = 500
_NEG = -3.0e38


def _acos(x):
    return jnp.arctan2(jnp.sqrt(jnp.maximum((1.0 - x) * (1.0 + x), 0.0)), x)


def _pass_a_body(C, CB, labels_ref, x_ref, out_ref, vals_ref, accmax, acccos):
    j = pl.program_id(0)
    x = x_ref[...]
    out_ref[...] = x * S
    lbl = labels_ref[0, :, 0, :]
    safe = jnp.where(lbl == -1, 0, lbl)
    cls = (lax.broadcasted_iota(jnp.int32, (_KR, 8, 8, 128), 0) + j * _KR) * 8 \
        + lax.broadcasted_iota(jnp.int32, (_KR, 8, 8, 128), 2)
    is_lbl = cls == safe[None, :, None, :]
    invalid = cls >= C
    mx = jnp.max(jnp.where(is_lbl | invalid, _NEG, x), axis=(0, 2))
    cs = jnp.max(jnp.where(is_lbl, x, _NEG), axis=(0, 2))

    @pl.when(j == 0)
    def _():
        accmax[...] = mx
        acccos[...] = cs

    @pl.when(j > 0)
    def _():
        accmax[...] = jnp.maximum(accmax[...], mx)
        acccos[...] = jnp.maximum(acccos[...], cs)

    @pl.when(j == CB - 1)
    def _():
        cos_y = acccos[...]
        max_o = accmax[...]
        ty = _acos(jnp.clip(cos_y, -1.0, 1.0))
        tm = _acos(jnp.clip(max_o, -1.0, 1.0))
        h = jnp.clip(jnp.float32(math.pi / 2) - (tm - ty),
                     0.0, jnp.float32(math.pi / 3))
        m_i = M2 + ALPHA * h
        phi = jnp.cos(M1 * ty + m_i) - M3
        fin = jnp.where(phi < cos_y, phi, cos_y)
        vals_ref[...] = jnp.where(lbl == -1, cos_y, fin) * S


def _make_pass_a(RT, C, interpret=False):
    CB = (RT + _KR - 1) // _KR
    return pl.pallas_call(
        functools.partial(_pass_a_body, C, CB),
        grid=(CB,),
        in_specs=[
            pl.BlockSpec((1, 8, 1, 128), lambda j: (0, 0, 0, 0)),
            pl.BlockSpec((_KR, 8, 8, 128), lambda j: (j, 0, 0, 0)),
        ],
        out_specs=[
            pl.BlockSpec((_KR, 8, 8, 128), lambda j: (j, 0, 0, 0)),
            pl.BlockSpec((8, 128), lambda j: (0, 0)),
        ],
        out_shape=[
            jax.ShapeDtypeStruct((RT, 8, 8, 128), jnp.float32),
            jax.ShapeDtypeStruct((8, 128), jnp.float32),
        ],
        scratch_shapes=[
            pltpu.VMEM((8, 128), jnp.float32),
            pltpu.VMEM((8, 128), jnp.float32),
        ],
        compiler_params=pltpu.CompilerParams(
            dimension_semantics=("arbitrary",),
            vmem_limit_bytes=100 * 1024 * 1024,
        ),
        interpret=interpret,
    )


def _make_sc_scatter(B):
    info = plsc.get_sparse_core_info()
    NC, NS = info.num_cores, info.num_subcores
    nw = NC * NS
    per_w = B // nw
    mesh = plsc.VectorSubcoreMesh(core_axis_name="c", subcore_axis_name="s")

    @functools.partial(
        pl.kernel,
        mesh=mesh,
        out_type=(),
        scratch_types=[
            pltpu.VMEM((per_w,), jnp.int32),
            pltpu.VMEM((per_w,), jnp.float32),
            pltpu.SemaphoreType.DMA,
        ],
    )
    def scatter(out_hbm, vals_hbm, idx_hbm, idx_v, vals_v, sem):
        wid = lax.axis_index("s") * NC + lax.axis_index("c")
        base = wid * per_w
        pltpu.sync_copy(idx_hbm.at[pl.ds(base, per_w)], idx_v)
        pltpu.sync_copy(vals_hbm.at[pl.ds(base, per_w)], vals_v)
        pltpu.async_copy(vals_v, out_hbm.at[idx_v], sem).wait()

    return scatter


def kernel(logits, labels):
    B, C = logits.shape
    RT = C // 8
    x4 = logits.T.reshape(RT, 8, B // 128, 128).transpose(0, 2, 1, 3)
    lbl4 = labels.reshape(1, B // 128, 1, 128)
    out4, vals2 = _make_pass_a(RT, C)(lbl4, x4)
    vals = vals2.reshape(B)

    safe = jnp.where(labels == -1, 0, labels)
    b = jnp.arange(B, dtype=jnp.int32)
    idx = (safe // 8) * (B * 8) + (b // 128) * 1024 + (safe % 8) * 128 + (b % 128)
    out_ref = jax.new_ref(out4.reshape(RT * 8 * B))
    _make_sc_scatter(B)(out_ref, vals, idx)
    return out_ref[...].reshape(RT, B // 128, 8, 128).transpose(0, 2, 1, 3).reshape(C, B).T
